# baseline (speedup 1.0000x reference)
import functools

import jax
import jax.numpy as jnp
from jax.experimental import pallas as pl
from jax.experimental.pallas import tpu as pltpu
from jax.experimental.pallas import tpu_sc as plsc

BS = 4; N = 256; VOCAB = 32000; D = 768
B = 4; C = 8; G = B * C; D_COL = 64; D_MEM = 64
R_SLOTS = 128; M = 2048; C_EM = 16
BSB = BS * B; TAU = 1.0
T = BS * N
TM = N * C
VT = 1280
EPT = 32
SCALE = 1.0 / (D_MEM ** 0.5)
F32 = jnp.float32


def _mm(a, b):
    return jax.lax.dot_general(a, b, (((1,), (0,)), ((), ())),
                               preferred_element_type=F32)


def _mmT(a, b):
    return jax.lax.dot_general(a, b, (((1,), (1,)), ((), ())),
                               preferred_element_type=F32)


def _mTm(a, b):
    return jax.lax.dot_general(a, b, (((0,), (0,)), ((), ())),
                               preferred_element_type=F32)


def _exp_unnorm(s):
    p = jnp.exp(s - jnp.max(s, axis=-1, keepdims=True))
    return p, 1.0 / jnp.sum(p, axis=-1, keepdims=True)


def _topk_onehots(vals, kk):
    L = vals.shape[1]
    iota = jax.lax.broadcasted_iota(jnp.int32, (1, L), 1)
    row_iota = jax.lax.broadcasted_iota(jnp.int32, (kk, 1), 0)

    def body(i, carry):
        v, O = carry
        mval = jnp.max(v)
        idx = jnp.min(jnp.where(v == mval, iota, L))
        onehot = (iota == idx).astype(F32)
        rowsel = (row_iota == i).astype(F32)
        O = O + rowsel * onehot
        v = jnp.where(iota == idx, -jnp.inf, v)
        return v, O

    _, O = jax.lax.fori_loop(0, kk, body, (vals, jnp.zeros((kk, L), F32)))
    return O


def _embed(input_ids, emb):
    ids = input_ids.reshape(T).astype(jnp.int32)
    info = plsc.get_sparse_core_info()
    nw = info.num_cores * info.num_subcores
    bpw = T // nw
    mesh = plsc.VectorSubcoreMesh(core_axis_name="c", subcore_axis_name="s")

    @functools.partial(
        pl.kernel, mesh=mesh,
        out_type=jax.ShapeDtypeStruct((T, D), F32),
        scratch_types=[
            pltpu.VMEM((bpw,), jnp.int32),
            pltpu.VMEM((bpw, D), F32),
            pltpu.SemaphoreType.DMA,
        ],
    )
    def k(table_hbm, idx_hbm, out_hbm, idx_v, rows_v, sem):
        wid = jax.lax.axis_index("s") * info.num_cores + jax.lax.axis_index("c")
        base = wid * bpw
        pltpu.sync_copy(idx_hbm.at[pl.ds(base, bpw)], idx_v)
        pltpu.async_copy(table_hbm.at[idx_v], rows_v, sem).wait()
        pltpu.sync_copy(rows_v, out_hbm.at[pl.ds(base, bpw)])

    return k(emb, ids)


def _mega_kernel(x_ref, pos_ref, Wfo_ref, bfo_ref, Wc_ref, bc_ref, Wk_ref,
                 Wv_ref, Wg_ref, Wq_ref, Wvn_ref, Wnp_ref, Wrp_ref, Wre_ref,
                 pmK_ref, pmV_ref, pma_ref, emK_ref, emV_ref, emS_ref,
                 wpm_ref, wem_ref, lam_ref, xf_ref):
    x = x_ref[...] + pos_ref[...]
    xflat = _mm(x, Wfo_ref[...]) + bfo_ref[...]
    Wrp = Wrp_ref[...]
    Wre = Wre_ref[...]
    pmK = pmK_ref[0]
    pmV = pmV_ref[0]
    emK = emK_ref[0]
    emV = emV_ref[0]
    nk = jnp.sqrt(jnp.sum(emK * emK, axis=-1)) + 1e-6
    rk = 1.0 / (SCALE * nk)

    xo_l = []
    q_l = []
    vn_l = []
    nov_l = []
    eligK = jnp.zeros((R_SLOTS, D_MEM), F32)
    eligV = jnp.zeros((R_SLOTS, D_MEM), F32)
    for c in range(C):
        xc = xflat[:, c * D_COL:(c + 1) * D_COL]
        h = jnp.tanh(_mm(xc, Wc_ref[c]) + bc_ref[0, c])
        k = _mm(xc, Wk_ref[c])
        v = _mm(xc, Wv_ref[c])
        q = _mm(xc, Wq_ref[c])
        vnc = _mm(xc, Wvn_ref[c])
        gate = jax.nn.sigmoid(
            jnp.sum(xc * Wg_ref[0, c], axis=-1, keepdims=True))
        wnc = jax.nn.sigmoid(
            jnp.sum(xc * Wnp_ref[0, c], axis=-1, keepdims=True))
        nq = jnp.sqrt(jnp.sum(q * q, axis=-1, keepdims=True)) + 1e-6
        qs = q * SCALE
        pp, rp = _exp_unnorm(_mmT(qs, pmK))
        pr = _mm(pp, pmV) * rp
        se = _mmT(qs, emK)
        pe, re = _exp_unnorm(se)
        er = _mm(pe, emV) * re
        msc = jnp.max(se * rk[None, :], axis=-1, keepdims=True) / nq
        xo = h + _mm(pr, Wrp) + _mm(er, Wre)
        d = xo - xc
        surp = jnp.sqrt(jnp.sum(d * d, axis=-1, keepdims=True))
        nov_l.append(surp * wnc * (1.0 - msc))
        kn = k / (jnp.sqrt(jnp.sum(k * k, axis=-1, keepdims=True)) + 1e-6)
        pg, rg = _exp_unnorm(_mmT(kn, pmK) * (1.0 / TAU))
        gr = pg * (gate * rg)
        eligK = eligK + _mTm(gr, k)
        eligV = eligV + _mTm(gr, v)
        xo_l.append(xo)
        q_l.append(q)
        vn_l.append(vnc)

    enorm = jnp.sqrt(jnp.sum(eligK * eligK, axis=-1))
    wpm = wpm_ref[0]
    gpm = jax.nn.sigmoid(jnp.mean(enorm) * wpm[0]
                         + 0.99 * jnp.sum(pma_ref[0]) * wpm[1]
                         + jnp.sum(jnp.mean(eligK, axis=0) * wpm[2:]))
    pm1K = pmK + gpm * eligK
    pm1V = pmV + gpm * eligV

    nov = jnp.concatenate(nov_l, axis=1)
    iota_n = jax.lax.broadcasted_iota(jnp.int32, (N, C), 0)
    iota_c = jax.lax.broadcasted_iota(jnp.int32, (N, C), 1)
    fidx = iota_n * C + iota_c
    row16 = jax.lax.broadcasted_iota(jnp.int32, (C_EM, 1), 0)

    def tk_body(i, carry):
        novv, candK, candV, scores = carry
        mval = jnp.max(novv)
        idx = jnp.min(jnp.where(novv == mval, fidx, TM))
        onehot = (fidx == idx).astype(F32)
        qsel = jnp.zeros((1, D_MEM), F32)
        vsel = jnp.zeros((1, D_MEM), F32)
        for c in range(C):
            col = onehot[:, c:c + 1]
            qsel = qsel + jnp.sum(col * q_l[c], axis=0, keepdims=True)
            vsel = vsel + jnp.sum(col * vn_l[c], axis=0, keepdims=True)
        rowsel = (row16 == i).astype(F32)
        candK = candK + rowsel * qsel
        candV = candV + rowsel * vsel
        scores = scores + rowsel * mval
        novv = jnp.where(fidx == idx, -jnp.inf, novv)
        return novv, candK, candV, scores

    z16 = jnp.zeros((C_EM, D_MEM), F32)
    _, candK, candV, scores = jax.lax.fori_loop(
        0, C_EM, tk_body, (nov, z16, z16, jnp.zeros((C_EM, 1), F32)))

    emS = emS_ref[0]
    Os = _topk_onehots(-emS, C_EM)
    oldK = _mm(Os, emK)
    oldV = _mm(Os, emV)
    wem = wem_ref[0]
    gem = jax.nn.sigmoid(jnp.mean(scores) * wem[0]
                         + jnp.sum(emS) * wem[1]
                         + jnp.sum(jnp.mean(candK, axis=0) * wem[2:]))
    em1K = emK + _mTm(Os, gem * (candK - oldK))
    em1V = emV + _mTm(Os, gem * (candV - oldV))

    lam = jax.nn.sigmoid(lam_ref[0, 0])
    xf_l = []
    for c in range(C):
        xc = xo_l[c]
        h2 = jnp.tanh(_mm(xc, Wc_ref[c]) + bc_ref[0, c])
        q2s = _mm(xc, Wq_ref[c]) * SCALE
        pp2, rp2 = _exp_unnorm(_mmT(q2s, pm1K))
        pr2 = _mm(pp2, pm1V) * rp2
        pe2, re2 = _exp_unnorm(_mmT(q2s, em1K))
        er2 = _mm(pe2, em1V) * re2
        xo2 = h2 + _mm(pr2, Wrp) + _mm(er2, Wre)
        xf_l.append((1.0 - lam) * xc + lam * xo2)
    xf_ref[...] = jnp.concatenate(xf_l, axis=1)


def _mega(x, pos_emb, p, lam_logit):
    CW = C * D_COL
    wspec = lambda shp: pl.BlockSpec(shp, lambda i, j: (0, 0))
    return pl.pallas_call(
        _mega_kernel,
        grid=(B, BS),
        in_specs=[
            pl.BlockSpec((N, D), lambda i, j: (j, 0)),
            pl.BlockSpec((N, D), lambda i, j: (0, 0)),
            pl.BlockSpec((D, CW), lambda i, j: (0, i)),
            pl.BlockSpec((1, CW), lambda i, j: (0, i)),
            pl.BlockSpec((C, D_COL, D_COL), lambda i, j: (i, 0, 0)),
            pl.BlockSpec((1, C, D_COL), lambda i, j: (i, 0, 0)),
            pl.BlockSpec((C, D_COL, D_MEM), lambda i, j: (i, 0, 0)),
            pl.BlockSpec((C, D_COL, D_MEM), lambda i, j: (i, 0, 0)),
            pl.BlockSpec((1, C, D_COL), lambda i, j: (i, 0, 0)),
            pl.BlockSpec((C, D_COL, D_MEM), lambda i, j: (i, 0, 0)),
            pl.BlockSpec((C, D_COL, D_MEM), lambda i, j: (i, 0, 0)),
            pl.BlockSpec((1, C, D_COL), lambda i, j: (i, 0, 0)),
            wspec((D_MEM, D_COL)),
            wspec((D_MEM, D_COL)),
            pl.BlockSpec((1, R_SLOTS, D_MEM), lambda i, j: (j * B + i, 0, 0)),
            pl.BlockSpec((1, R_SLOTS, D_MEM), lambda i, j: (j * B + i, 0, 0)),
            pl.BlockSpec((1, 1, R_SLOTS), lambda i, j: (j * B + i, 0, 0)),
            pl.BlockSpec((1, M, D_MEM), lambda i, j: (j * B + i, 0, 0)),
            pl.BlockSpec((1, M, D_MEM), lambda i, j: (j * B + i, 0, 0)),
            pl.BlockSpec((1, 1, M), lambda i, j: (j * B + i, 0, 0)),
            wspec((1, D_MEM + 2)),
            wspec((1, D_MEM + 2)),
            wspec((1, 1)),
        ],
        out_specs=pl.BlockSpec((N, CW), lambda i, j: (j, i)),
        out_shape=jax.ShapeDtypeStruct((T, G * D_COL), F32),
    )(x, pos_emb, p["W_fan_out"], p["b_fan_out"].reshape(1, G * D_COL),
      p["W_col"], p["b_col"].reshape(B, C, D_COL), p["W_k"], p["W_v"],
      p["w_gate"].reshape(B, C, D_COL), p["W_q"], p["W_vn"],
      p["w_nov_proj"].reshape(B, C, D_COL), p["W_read_pm"], p["W_read_em"],
      p["pm_K"], p["pm_V"], p["pm_a"].reshape(BSB, 1, R_SLOTS),
      p["em_K"], p["em_V"], p["em_S"].reshape(BSB, 1, M),
      p["w_pm_mod"].reshape(1, D_MEM + 2), p["w_em_mod"].reshape(1, D_MEM + 2),
      lam_logit.reshape(1, 1))


def _fanin_ln_kernel(x_ref, w_ref, b_ref, g_ref, beta_ref, o_ref):
    y = jnp.dot(x_ref[...], w_ref[...], preferred_element_type=F32) + b_ref[...]
    m = jnp.mean(y, axis=-1, keepdims=True)
    v = jnp.mean((y - m) * (y - m), axis=-1, keepdims=True)
    o_ref[...] = (y - m) * jax.lax.rsqrt(v + 1e-5) * g_ref[...] + beta_ref[...]


def _fan_in_ln(x, W, b, g, beta):
    return pl.pallas_call(
        _fanin_ln_kernel,
        grid=(BS,),
        in_specs=[
            pl.BlockSpec((N, G * D_COL), lambda i: (i, 0)),
            pl.BlockSpec((G * D_COL, D), lambda i: (0, 0)),
            pl.BlockSpec((1, D), lambda i: (0, 0)),
            pl.BlockSpec((1, D), lambda i: (0, 0)),
            pl.BlockSpec((1, D), lambda i: (0, 0)),
        ],
        out_specs=pl.BlockSpec((N, D), lambda i: (i, 0)),
        out_shape=jax.ShapeDtypeStruct((T, D), F32),
    )(x, W, b.reshape(1, D), g.reshape(1, D), beta.reshape(1, D))


def _logits_kernel(x_ref, e_ref, o_ref):
    o_ref[...] = jax.lax.dot_general(x_ref[...], e_ref[...],
                                     (((1,), (1,)), ((), ())),
                                     preferred_element_type=F32)


def _logits(x, emb):
    return pl.pallas_call(
        _logits_kernel,
        grid=(VOCAB // VT,),
        in_specs=[
            pl.BlockSpec((T, D), lambda j: (0, 0)),
            pl.BlockSpec((VT, D), lambda j: (j, 0)),
        ],
        out_specs=pl.BlockSpec((T, VT), lambda j: (0, j)),
        out_shape=jax.ShapeDtypeStruct((T, VOCAB), F32),
    )(x, emb)


def kernel(input_ids, emb, pos_emb, W_fan_out, b_fan_out, W_col, b_col, W_k,
           W_v, w_gate, W_q, W_vn, w_nov_proj, W_read_pm, W_read_em, pm_K,
           pm_V, pm_a, em_K, em_V, em_S, w_pm_mod, w_em_mod, W_fan_in,
           b_fan_in, ln_g, ln_b, lambda_logit):
    p = dict(W_fan_out=W_fan_out, b_fan_out=b_fan_out, W_col=W_col,
             b_col=b_col, W_k=W_k, W_v=W_v, w_gate=w_gate, W_q=W_q,
             W_vn=W_vn, w_nov_proj=w_nov_proj, W_read_pm=W_read_pm,
             W_read_em=W_read_em, pm_K=pm_K, pm_V=pm_V, pm_a=pm_a,
             em_K=em_K, em_V=em_V, em_S=em_S, w_pm_mod=w_pm_mod,
             w_em_mod=w_em_mod)
    x = _embed(input_ids, emb)
    xf = _mega(x, pos_emb, p, lambda_logit)
    xn = _fan_in_ln(xf, W_fan_in, b_fan_in, ln_g, ln_b)
    logits = _logits(xn, emb).reshape(BS, N, VOCAB)
    return (logits, jnp.array(0.0, F32))


# speedup vs baseline: 1.1428x; 1.1428x over previous
import functools

import jax
import jax.numpy as jnp
from jax.experimental import pallas as pl
from jax.experimental.pallas import tpu as pltpu
from jax.experimental.pallas import tpu_sc as plsc

BS = 4; N = 256; VOCAB = 32000; D = 768
B = 4; C = 8; G = B * C; D_COL = 64; D_MEM = 64
R_SLOTS = 128; M = 2048; C_EM = 16
BSB = BS * B; TAU = 1.0
T = BS * N
TM = N * C
VT = 1280
EPT = 32
SCALE = 1.0 / (D_MEM ** 0.5)
F32 = jnp.float32


def _mm(a, b):
    return jax.lax.dot_general(a, b, (((1,), (0,)), ((), ())),
                               preferred_element_type=F32)


def _mmT(a, b):
    return jax.lax.dot_general(a, b, (((1,), (1,)), ((), ())),
                               preferred_element_type=F32)


def _mTm(a, b):
    return jax.lax.dot_general(a, b, (((0,), (0,)), ((), ())),
                               preferred_element_type=F32)


BF16 = jnp.bfloat16


def _self_max(s):
    return s, jnp.max(s, axis=-1, keepdims=True)


def _attn_read(qs, K_b, V_aug_b):
    se = _mmT(qs.astype(BF16), K_b).astype(BF16)
    mx = jnp.max(se, axis=-1, keepdims=True)
    pe = jnp.exp(se.astype(F32) - mx.astype(F32)).astype(BF16)
    ra = _mm(pe, V_aug_b)
    read = ra[:, :D_MEM] * (1.0 / ra[:, D_MEM:D_MEM + 1])
    return read, se


def _topk_onehots(vals, kk):
    L = vals.shape[1]
    iota = jax.lax.broadcasted_iota(jnp.int32, (1, L), 1)
    row_iota = jax.lax.broadcasted_iota(jnp.int32, (kk, 1), 0)

    def body(i, carry):
        v, O = carry
        mval = jnp.max(v)
        idx = jnp.min(jnp.where(v == mval, iota, L))
        onehot = (iota == idx).astype(F32)
        rowsel = (row_iota == i).astype(F32)
        O = O + rowsel * onehot
        v = jnp.where(iota == idx, -jnp.inf, v)
        return v, O

    _, O = jax.lax.fori_loop(0, kk, body, (vals, jnp.zeros((kk, L), F32)))
    return O


def _embed(input_ids, emb):
    ids = input_ids.reshape(T).astype(jnp.int32)
    info = plsc.get_sparse_core_info()
    nw = info.num_cores * info.num_subcores
    bpw = T // nw
    mesh = plsc.VectorSubcoreMesh(core_axis_name="c", subcore_axis_name="s")

    @functools.partial(
        pl.kernel, mesh=mesh,
        out_type=jax.ShapeDtypeStruct((T, D), F32),
        scratch_types=[
            pltpu.VMEM((bpw,), jnp.int32),
            pltpu.VMEM((bpw, D), F32),
            pltpu.SemaphoreType.DMA,
        ],
    )
    def k(table_hbm, idx_hbm, out_hbm, idx_v, rows_v, sem):
        wid = jax.lax.axis_index("s") * info.num_cores + jax.lax.axis_index("c")
        base = wid * bpw
        pltpu.sync_copy(idx_hbm.at[pl.ds(base, bpw)], idx_v)
        pltpu.async_copy(table_hbm.at[idx_v], rows_v, sem).wait()
        pltpu.sync_copy(rows_v, out_hbm.at[pl.ds(base, bpw)])

    return k(emb, ids)


def _mega_kernel(x_ref, pos_ref, Wfo_ref, bfo_ref, Wc_ref, bc_ref, Wk_ref,
                 Wv_ref, Wg_ref, Wq_ref, Wvn_ref, Wnp_ref, Wrp_ref, Wre_ref,
                 pmK_ref, pmV_ref, pma_ref, emK_ref, emV_ref, emS_ref,
                 wpm_ref, wem_ref, lam_ref, xf_ref):
    x = x_ref[...] + pos_ref[...]
    xflat = _mm(x, Wfo_ref[...]) + bfo_ref[...]
    Wrp = Wrp_ref[...]
    Wre = Wre_ref[...]
    pmK = pmK_ref[0]
    pmV = pmV_ref[0]
    emK = emK_ref[0]
    emV = emV_ref[0]
    nk = jnp.sqrt(jnp.sum(emK * emK, axis=-1)) + 1e-6
    rk = 1.0 / (SCALE * nk)
    onesM = jnp.ones((M, 1), F32)
    ones_pm = jnp.ones((R_SLOTS, 1), F32)
    emK_b = emK.astype(BF16)
    emV_aug_b = jnp.concatenate([emV, onesM], axis=1).astype(BF16)
    pmV_aug = jnp.concatenate([pmV, ones_pm], axis=1)

    xo_l = []
    q_l = []
    vn_l = []
    nov_l = []
    eligK = jnp.zeros((R_SLOTS, D_MEM), F32)
    eligV = jnp.zeros((R_SLOTS, D_MEM), F32)
    for c in range(C):
        xc = xflat[:, c * D_COL:(c + 1) * D_COL]
        h = jnp.tanh(_mm(xc, Wc_ref[c]) + bc_ref[0, c])
        k = _mm(xc, Wk_ref[c])
        v = _mm(xc, Wv_ref[c])
        q = _mm(xc, Wq_ref[c])
        vnc = _mm(xc, Wvn_ref[c])
        gate = jax.nn.sigmoid(
            jnp.sum(xc * Wg_ref[0, c], axis=-1, keepdims=True))
        wnc = jax.nn.sigmoid(
            jnp.sum(xc * Wnp_ref[0, c], axis=-1, keepdims=True))
        nq = jnp.sqrt(jnp.sum(q * q, axis=-1, keepdims=True)) + 1e-6
        qs = q * SCALE
        pp = jnp.exp(jnp.subtract(*_self_max(_mmT(qs, pmK))))
        pa = _mm(pp, pmV_aug)
        pr = pa[:, :D_MEM] * (1.0 / pa[:, D_MEM:D_MEM + 1])
        er, se = _attn_read(qs, emK_b, emV_aug_b)
        msc = jnp.max(se * rk[None, :], axis=-1, keepdims=True) / nq
        xo = h + _mm(pr, Wrp) + _mm(er, Wre)
        d = xo - xc
        surp = jnp.sqrt(jnp.sum(d * d, axis=-1, keepdims=True))
        nov_l.append(surp * wnc * (1.0 - msc))
        kn = k / (jnp.sqrt(jnp.sum(k * k, axis=-1, keepdims=True)) + 1e-6)
        pg = jnp.exp(jnp.subtract(*_self_max(_mmT(kn, pmK) * (1.0 / TAU))))
        gg = gate * (1.0 / jnp.sum(pg, axis=-1, keepdims=True))
        eligK = eligK + _mTm(pg, k * gg)
        eligV = eligV + _mTm(pg, v * gg)
        xo_l.append(xo)
        q_l.append(q)
        vn_l.append(vnc)

    enorm = jnp.sqrt(jnp.sum(eligK * eligK, axis=-1))
    wpm = wpm_ref[0]
    gpm = jax.nn.sigmoid(jnp.mean(enorm) * wpm[0]
                         + 0.99 * jnp.sum(pma_ref[0]) * wpm[1]
                         + jnp.sum(jnp.mean(eligK, axis=0) * wpm[2:]))
    pm1K = pmK + gpm * eligK
    pm1V = pmV + gpm * eligV

    nov = jnp.concatenate(nov_l, axis=1)
    iota_n = jax.lax.broadcasted_iota(jnp.int32, (N, C), 0)
    iota_c = jax.lax.broadcasted_iota(jnp.int32, (N, C), 1)
    fidx = iota_n * C + iota_c
    row16 = jax.lax.broadcasted_iota(jnp.int32, (C_EM, 1), 0)

    def tk_body(i, carry):
        novv, candK, candV, scores = carry
        mval = jnp.max(novv)
        idx = jnp.min(jnp.where(novv == mval, fidx, TM))
        onehot = (fidx == idx).astype(F32)
        qsel = jnp.zeros((1, D_MEM), F32)
        vsel = jnp.zeros((1, D_MEM), F32)
        for c in range(C):
            col = onehot[:, c:c + 1]
            qsel = qsel + jnp.sum(col * q_l[c], axis=0, keepdims=True)
            vsel = vsel + jnp.sum(col * vn_l[c], axis=0, keepdims=True)
        rowsel = (row16 == i).astype(F32)
        candK = candK + rowsel * qsel
        candV = candV + rowsel * vsel
        scores = scores + rowsel * mval
        novv = jnp.where(fidx == idx, -jnp.inf, novv)
        return novv, candK, candV, scores

    z16 = jnp.zeros((C_EM, D_MEM), F32)
    _, candK, candV, scores = jax.lax.fori_loop(
        0, C_EM, tk_body, (nov, z16, z16, jnp.zeros((C_EM, 1), F32)))

    emS = emS_ref[0]
    Os = _topk_onehots(-emS, C_EM)
    oldK = _mm(Os, emK)
    oldV = _mm(Os, emV)
    wem = wem_ref[0]
    gem = jax.nn.sigmoid(jnp.mean(scores) * wem[0]
                         + jnp.sum(emS) * wem[1]
                         + jnp.sum(jnp.mean(candK, axis=0) * wem[2:]))
    em1K = emK + _mTm(Os, gem * (candK - oldK))
    em1V = emV + _mTm(Os, gem * (candV - oldV))

    em1K_b = em1K.astype(BF16)
    em1V_aug_b = jnp.concatenate([em1V, onesM], axis=1).astype(BF16)
    pm1V_aug = jnp.concatenate([pm1V, ones_pm], axis=1)
    lam = jax.nn.sigmoid(lam_ref[0, 0])
    xf_l = []
    for c in range(C):
        xc = xo_l[c]
        h2 = jnp.tanh(_mm(xc, Wc_ref[c]) + bc_ref[0, c])
        q2s = _mm(xc, Wq_ref[c]) * SCALE
        pp2 = jnp.exp(jnp.subtract(*_self_max(_mmT(q2s, pm1K))))
        pa2 = _mm(pp2, pm1V_aug)
        pr2 = pa2[:, :D_MEM] * (1.0 / pa2[:, D_MEM:D_MEM + 1])
        er2, _ = _attn_read(q2s, em1K_b, em1V_aug_b)
        xo2 = h2 + _mm(pr2, Wrp) + _mm(er2, Wre)
        xf_l.append((1.0 - lam) * xc + lam * xo2)
    xf_ref[...] = jnp.concatenate(xf_l, axis=1)


def _mega(x, pos_emb, p, lam_logit):
    CW = C * D_COL
    wspec = lambda shp: pl.BlockSpec(shp, lambda i, j: (0, 0))
    return pl.pallas_call(
        _mega_kernel,
        grid=(B, BS),
        in_specs=[
            pl.BlockSpec((N, D), lambda i, j: (j, 0)),
            pl.BlockSpec((N, D), lambda i, j: (0, 0)),
            pl.BlockSpec((D, CW), lambda i, j: (0, i)),
            pl.BlockSpec((1, CW), lambda i, j: (0, i)),
            pl.BlockSpec((C, D_COL, D_COL), lambda i, j: (i, 0, 0)),
            pl.BlockSpec((1, C, D_COL), lambda i, j: (i, 0, 0)),
            pl.BlockSpec((C, D_COL, D_MEM), lambda i, j: (i, 0, 0)),
            pl.BlockSpec((C, D_COL, D_MEM), lambda i, j: (i, 0, 0)),
            pl.BlockSpec((1, C, D_COL), lambda i, j: (i, 0, 0)),
            pl.BlockSpec((C, D_COL, D_MEM), lambda i, j: (i, 0, 0)),
            pl.BlockSpec((C, D_COL, D_MEM), lambda i, j: (i, 0, 0)),
            pl.BlockSpec((1, C, D_COL), lambda i, j: (i, 0, 0)),
            wspec((D_MEM, D_COL)),
            wspec((D_MEM, D_COL)),
            pl.BlockSpec((1, R_SLOTS, D_MEM), lambda i, j: (j * B + i, 0, 0)),
            pl.BlockSpec((1, R_SLOTS, D_MEM), lambda i, j: (j * B + i, 0, 0)),
            pl.BlockSpec((1, 1, R_SLOTS), lambda i, j: (j * B + i, 0, 0)),
            pl.BlockSpec((1, M, D_MEM), lambda i, j: (j * B + i, 0, 0)),
            pl.BlockSpec((1, M, D_MEM), lambda i, j: (j * B + i, 0, 0)),
            pl.BlockSpec((1, 1, M), lambda i, j: (j * B + i, 0, 0)),
            wspec((1, D_MEM + 2)),
            wspec((1, D_MEM + 2)),
            wspec((1, 1)),
        ],
        out_specs=pl.BlockSpec((N, CW), lambda i, j: (j, i)),
        out_shape=jax.ShapeDtypeStruct((T, G * D_COL), F32),
    )(x, pos_emb, p["W_fan_out"], p["b_fan_out"].reshape(1, G * D_COL),
      p["W_col"], p["b_col"].reshape(B, C, D_COL), p["W_k"], p["W_v"],
      p["w_gate"].reshape(B, C, D_COL), p["W_q"], p["W_vn"],
      p["w_nov_proj"].reshape(B, C, D_COL), p["W_read_pm"], p["W_read_em"],
      p["pm_K"], p["pm_V"], p["pm_a"].reshape(BSB, 1, R_SLOTS),
      p["em_K"], p["em_V"], p["em_S"].reshape(BSB, 1, M),
      p["w_pm_mod"].reshape(1, D_MEM + 2), p["w_em_mod"].reshape(1, D_MEM + 2),
      lam_logit.reshape(1, 1))


def _fanin_ln_kernel(x_ref, w_ref, b_ref, g_ref, beta_ref, o_ref):
    y = jnp.dot(x_ref[...], w_ref[...], preferred_element_type=F32) + b_ref[...]
    m = jnp.mean(y, axis=-1, keepdims=True)
    v = jnp.mean((y - m) * (y - m), axis=-1, keepdims=True)
    o_ref[...] = (y - m) * jax.lax.rsqrt(v + 1e-5) * g_ref[...] + beta_ref[...]


def _fan_in_ln(x, W, b, g, beta):
    return pl.pallas_call(
        _fanin_ln_kernel,
        grid=(BS,),
        in_specs=[
            pl.BlockSpec((N, G * D_COL), lambda i: (i, 0)),
            pl.BlockSpec((G * D_COL, D), lambda i: (0, 0)),
            pl.BlockSpec((1, D), lambda i: (0, 0)),
            pl.BlockSpec((1, D), lambda i: (0, 0)),
            pl.BlockSpec((1, D), lambda i: (0, 0)),
        ],
        out_specs=pl.BlockSpec((N, D), lambda i: (i, 0)),
        out_shape=jax.ShapeDtypeStruct((T, D), F32),
    )(x, W, b.reshape(1, D), g.reshape(1, D), beta.reshape(1, D))


def _logits_kernel(x_ref, e_ref, o_ref):
    o_ref[...] = jax.lax.dot_general(x_ref[...], e_ref[...],
                                     (((1,), (1,)), ((), ())),
                                     preferred_element_type=F32)


def _logits(x, emb):
    return pl.pallas_call(
        _logits_kernel,
        grid=(VOCAB // VT,),
        in_specs=[
            pl.BlockSpec((T, D), lambda j: (0, 0)),
            pl.BlockSpec((VT, D), lambda j: (j, 0)),
        ],
        out_specs=pl.BlockSpec((T, VT), lambda j: (0, j)),
        out_shape=jax.ShapeDtypeStruct((T, VOCAB), F32),
    )(x, emb)


def kernel(input_ids, emb, pos_emb, W_fan_out, b_fan_out, W_col, b_col, W_k,
           W_v, w_gate, W_q, W_vn, w_nov_proj, W_read_pm, W_read_em, pm_K,
           pm_V, pm_a, em_K, em_V, em_S, w_pm_mod, w_em_mod, W_fan_in,
           b_fan_in, ln_g, ln_b, lambda_logit):
    p = dict(W_fan_out=W_fan_out, b_fan_out=b_fan_out, W_col=W_col,
             b_col=b_col, W_k=W_k, W_v=W_v, w_gate=w_gate, W_q=W_q,
             W_vn=W_vn, w_nov_proj=w_nov_proj, W_read_pm=W_read_pm,
             W_read_em=W_read_em, pm_K=pm_K, pm_V=pm_V, pm_a=pm_a,
             em_K=em_K, em_V=em_V, em_S=em_S, w_pm_mod=w_pm_mod,
             w_em_mod=w_em_mod)
    x = _embed(input_ids, emb)
    xf = _mega(x, pos_emb, p, lambda_logit)
    xn = _fan_in_ln(xf, W_fan_in, b_fan_in, ln_g, ln_b)
    logits = _logits(xn, emb).reshape(BS, N, VOCAB)
    return (logits, jnp.array(0.0, F32))
